# Initial kernel scaffold; baseline (speedup 1.0000x reference)
import jax
import jax.numpy as jnp
from jax.experimental import pallas as pl


def kernel(cart, atomindex, shifts, species, emb_params, mp_params, out_params):
    raise NotImplementedError("write your pallas kernel here")


# speedup vs baseline: 39.9939x; 39.9939x over previous
import functools

import jax
import jax.numpy as jnp
from jax import lax
from jax.experimental import pallas as pl
from jax.experimental.pallas import tpu as pltpu
from jax.experimental.pallas import tpu_sc as plsc

NWAVE = 8
MAX_L = 2
R_MAX_L = MAX_L + 1
NSPH = R_MAX_L * R_MAX_L
NORBIT = NWAVE * R_MAX_L
NATOM = 10000
NEIGH = 320000
CUTOFF = 5.0

NC = 2
NS = 16
LANES = 16
NWORK = NC * NS
EW = NEIGH // NWORK
CHUNK = 80
NCHUNK = EW // CHUNK
SW = 80
TW0 = 32
TW = 80
ZROWS = 125

_f32 = jnp.float32


def _dot_like_ref(x, w):
    if w.shape[0] == 1:
        return x * w[0:1, :]
    return jnp.dot(x.astype(jnp.bfloat16), w.astype(jnp.bfloat16),
                   preferred_element_type=_f32)


def _mlp(x, params):
    for w, b in params[:-1]:
        x = jnp.tanh(_dot_like_ref(x, w[...]) + b[...])
    w, b = params[-1]
    return _dot_like_ref(x, w[...]) + b[...]


def _embed_body(sp_ref, w1, b1, w2, b2, w3, b3, out_ref):
    out_ref[...] = _mlp(sp_ref[...], [(w1, b1), (w2, b2), (w3, b3)])


def _embed(species_col, emb_params):
    br = 2000
    wspecs = []
    wvals = []
    for w, b in emb_params:
        wspecs += [pl.BlockSpec(w.shape, lambda i: (0, 0)),
                   pl.BlockSpec((1, b.shape[0]), lambda i: (0, 0))]
        wvals += [w, b.reshape(1, -1)]
    return pl.pallas_call(
        _embed_body,
        grid=(NATOM // br,),
        in_specs=[pl.BlockSpec((br, 1), lambda i: (i, 0))] + wspecs,
        out_specs=pl.BlockSpec((br, NORBIT), lambda i: (i, 0)),
        out_shape=jax.ShapeDtypeStruct((NATOM, NORBIT), _f32),
    )(species_col, *wvals)


def _dxyz_body(x_hbm, y_hbm, z_hbm, idx0_hbm, idx1_hbm, out_hbm,
               xt, yt, zt, i0b, i1b, ob):
    c = lax.axis_index("c")
    s = lax.axis_index("s")
    wid = c * NS + s
    pltpu.sync_copy(x_hbm, xt)
    pltpu.sync_copy(y_hbm, yt)
    pltpu.sync_copy(z_hbm, zt)

    def chunk_body(t, carry):
        base = wid * EW + t * CHUNK
        pltpu.sync_copy(idx0_hbm.at[pl.ds(base, CHUNK)], i0b.at[0])
        pltpu.sync_copy(idx1_hbm.at[pl.ds(base, CHUNK)], i1b.at[0])
        for e in range(CHUNK // LANES):
            i0 = i0b[0, pl.ds(e * LANES, LANES)]
            i1 = i1b[0, pl.ds(e * LANES, LANES)]
            for row, tab in ((0, xt), (1, yt), (2, zt)):
                a = plsc.load_gather(tab, [i0])
                b = plsc.load_gather(tab, [i1])
                ob[row, pl.ds(e * LANES, LANES)] = b - a
        pltpu.sync_copy(ob, out_hbm.at[:, pl.ds(base, CHUNK)])
        return carry

    lax.fori_loop(0, NCHUNK, chunk_body, 0)


def _dxyz(cart, idx0, idx1):
    mesh = plsc.VectorSubcoreMesh(core_axis_name="c", subcore_axis_name="s")
    f = pl.kernel(
        _dxyz_body,
        out_type=jax.ShapeDtypeStruct((3, NEIGH), _f32),
        mesh=mesh,
        compiler_params=pltpu.CompilerParams(use_tc_tiling_on_sc=False, needs_layout_passes=False),
        scratch_types=[
            pltpu.VMEM((NATOM,), _f32),
            pltpu.VMEM((NATOM,), _f32),
            pltpu.VMEM((NATOM,), _f32),
            pltpu.VMEM((1, CHUNK), jnp.int32),
            pltpu.VMEM((1, CHUNK), jnp.int32),
            pltpu.VMEM((3, CHUNK), _f32),
        ],
    )
    return f(cart[0], cart[1], cart[2], idx0, idx1)


def _geom_body(dxyz_ref, shifts_ref, g0_ref, g1_ref):
    co = dxyz_ref[...] + shifts_ref[...]
    x = co[0:1, :]
    y = co[1:2, :]
    z = co[2:3, :]
    d = jnp.sqrt(x * x + y * y + z * z)
    tmp = (jnp.cos(d * (jnp.pi / CUTOFF)) + 1.0) * 0.5
    rcut = tmp * tmp * tmp
    xs = x * (1.0 / CUTOFF)
    ys = y * (1.0 / CUTOFF)
    zs = z * (1.0 / CUTOFF)
    c0 = 0.28209479177387814
    c1 = 0.4886025119029199
    c2 = 1.0925484305920792
    sph = jnp.concatenate([
        jnp.full_like(xs, c0),
        c1 * ys, c1 * zs, c1 * xs,
        c2 * xs * ys,
        c2 * ys * zs,
        0.31539156525252005 * (2.0 * zs * zs - xs * xs - ys * ys),
        c2 * xs * zs,
        0.5462742152960396 * (xs * xs - ys * ys),
    ], axis=0)
    g0_ref[...] = jnp.concatenate([d, rcut, sph], axis=0)
    g1_ref[...] = jnp.concatenate([rcut, sph], axis=0)


def _geom(dxyz, shifts):
    bn = 2560
    return pl.pallas_call(
        _geom_body,
        grid=(NEIGH // bn,),
        in_specs=[pl.BlockSpec((3, bn), lambda i: (0, i)),
                  pl.BlockSpec((3, bn), lambda i: (0, i))],
        out_specs=[pl.BlockSpec((11, bn), lambda i: (0, i)),
                   pl.BlockSpec((10, bn), lambda i: (0, i))],
        out_shape=[jax.ShapeDtypeStruct((11, NEIGH), _f32),
                   jax.ShapeDtypeStruct((10, NEIGH), _f32)],
    )(dxyz, shifts)


def _iota16():
    return lax.iota(jnp.int32, LANES)


def _pass0_body(idx0_hbm, idx1_hbm, g0_hbm, t0_hbm, s_hbm, gout_hbm,
                i0b, i1b, tb, db, gb, rb, zb, s_sh, sem):
    c = lax.axis_index("c")
    s = lax.axis_index("s")
    wid = c * NS + s

    zv = jnp.zeros((LANES,), _f32)

    def zfill(i, carry):
        for k in range(SW // LANES):
            zb[i, pl.ds(k * LANES, LANES)] = zv
        return carry

    lax.fori_loop(0, ZROWS, zfill, 0)
    rows0 = s * (NATOM // NS)
    for i in range(NATOM // NS // ZROWS):
        pltpu.sync_copy(zb, s_sh.at[pl.ds(rows0 + i * ZROWS, ZROWS)])

    def rfill(i, carry):
        for k in range(SW // LANES):
            rb[i, pl.ds(k * LANES, LANES)] = zv
        return carry

    lax.fori_loop(0, CHUNK, rfill, 0)
    plsc.subcore_barrier()

    def chunk_body(t, carry):
        base = wid * EW + t * CHUNK
        pltpu.sync_copy(idx0_hbm.at[pl.ds(base, CHUNK)], i0b.at[0])
        pltpu.sync_copy(idx1_hbm.at[pl.ds(base, CHUNK)], i1b.at[0])
        pltpu.sync_copy(g0_hbm.at[:, pl.ds(base, CHUNK)], db)
        pltpu.async_copy(t0_hbm.at[i1b.at[0]], tb, sem).wait()
        for e in range(CHUNK // LANES):
            off = e * LANES
            row = _iota16() + off
            d = db[0, pl.ds(off, LANES)]
            rc = db[1, pl.ds(off, LANES)]
            for j in range(NWAVE):
                cg = plsc.load_gather(tb, [row, jnp.full((LANES,), j, jnp.int32)])
                al = plsc.load_gather(tb, [row, jnp.full((LANES,), 8 + j, jnp.int32)])
                ce = plsc.load_gather(tb, [row, jnp.full((LANES,), 16 + j, jnp.int32)])
                sd = al * (d - ce)
                radial = jnp.exp(-(sd * sd))
                g = rc * radial
                gb[j, pl.ds(off, LANES)] = g
                w = g * cg
                for k in range(NSPH):
                    sphk = db[2 + k, pl.ds(off, LANES)]
                    plsc.store_scatter(
                        rb, [row, jnp.full((LANES,), k * NWAVE + j, jnp.int32)],
                        sphk * w)
        pltpu.sync_copy(gb, gout_hbm.at[:, pl.ds(base, CHUNK)])
        pltpu.sync_copy(rb, s_sh.at[i0b.at[0]], add=True)
        return carry

    lax.fori_loop(0, NCHUNK, chunk_body, 0)
    plsc.subcore_barrier()
    for i in range(NATOM // NS // ZROWS):
        r0 = rows0 + i * ZROWS
        pltpu.sync_copy(s_sh.at[pl.ds(r0, ZROWS)], s_hbm.at[c, pl.ds(r0, ZROWS)])


def _pass0(idx0, idx1, g0, t0):
    mesh = plsc.VectorSubcoreMesh(core_axis_name="c", subcore_axis_name="s")
    f = pl.kernel(
        _pass0_body,
        out_type=[jax.ShapeDtypeStruct((NC, NATOM, SW), _f32),
                  jax.ShapeDtypeStruct((NWAVE, NEIGH), _f32)],
        mesh=mesh,
        compiler_params=pltpu.CompilerParams(use_tc_tiling_on_sc=False, needs_layout_passes=False),
        scratch_types=[
            pltpu.VMEM((1, CHUNK), jnp.int32),
            pltpu.VMEM((1, CHUNK), jnp.int32),
            pltpu.VMEM((CHUNK, TW0), _f32),
            pltpu.VMEM((11, CHUNK), _f32),
            pltpu.VMEM((NWAVE, CHUNK), _f32),
            pltpu.VMEM((CHUNK, SW), _f32),
            pltpu.VMEM((ZROWS, SW), _f32),
            pltpu.VMEM_SHARED((NATOM, SW), _f32),
            pltpu.SemaphoreType.DMA,
        ],
    )
    return f(idx0, idx1, g0, t0)


def _passm_body(idx0_hbm, idx1_hbm, g1_hbm, gw_hbm, t_hbm, s_hbm,
                i0b, i1b, tb, db, gb, zb, s_sh, sem):
    c = lax.axis_index("c")
    s = lax.axis_index("s")
    wid = c * NS + s

    zv = jnp.zeros((LANES,), _f32)

    def zfill(i, carry):
        for k in range(SW // LANES):
            zb[i, pl.ds(k * LANES, LANES)] = zv
        return carry

    lax.fori_loop(0, ZROWS, zfill, 0)
    rows0 = s * (NATOM // NS)
    for i in range(NATOM // NS // ZROWS):
        pltpu.sync_copy(zb, s_sh.at[pl.ds(rows0 + i * ZROWS, ZROWS)])
    plsc.subcore_barrier()

    def chunk_body(t, carry):
        base = wid * EW + t * CHUNK
        pltpu.sync_copy(idx0_hbm.at[pl.ds(base, CHUNK)], i0b.at[0])
        pltpu.sync_copy(idx1_hbm.at[pl.ds(base, CHUNK)], i1b.at[0])
        pltpu.sync_copy(g1_hbm.at[:, pl.ds(base, CHUNK)], db)
        pltpu.sync_copy(gw_hbm.at[:, pl.ds(base, CHUNK)], gb)
        pltpu.async_copy(t_hbm.at[i1b.at[0]], tb, sem).wait()
        for e in range(CHUNK // LANES):
            off = e * LANES
            row = _iota16() + off
            rc = db[0, pl.ds(off, LANES)]
            ws = []
            for j in range(NWAVE):
                cg = plsc.load_gather(tb, [row, jnp.full((LANES,), 72 + j, jnp.int32)])
                ws.append(gb[j, pl.ds(off, LANES)] * cg)
            for k in range(NSPH):
                sphk = db[1 + k, pl.ds(off, LANES)]
                for j in range(NWAVE):
                    col = jnp.full((LANES,), k * NWAVE + j, jnp.int32)
                    mv = plsc.load_gather(tb, [row, col])
                    plsc.store_scatter(tb, [row, col], sphk * ws[j] + rc * mv)
        pltpu.sync_copy(tb, s_sh.at[i0b.at[0]], add=True)
        return carry

    lax.fori_loop(0, NCHUNK, chunk_body, 0)
    plsc.subcore_barrier()
    for i in range(NATOM // NS // ZROWS):
        r0 = rows0 + i * ZROWS
        pltpu.sync_copy(s_sh.at[pl.ds(r0, ZROWS)], s_hbm.at[c, pl.ds(r0, ZROWS)])


def _passm(idx0, idx1, g1, gw, t):
    mesh = plsc.VectorSubcoreMesh(core_axis_name="c", subcore_axis_name="s")
    f = pl.kernel(
        _passm_body,
        out_type=jax.ShapeDtypeStruct((NC, NATOM, SW), _f32),
        mesh=mesh,
        compiler_params=pltpu.CompilerParams(use_tc_tiling_on_sc=False, needs_layout_passes=False),
        scratch_types=[
            pltpu.VMEM((1, CHUNK), jnp.int32),
            pltpu.VMEM((1, CHUNK), jnp.int32),
            pltpu.VMEM((CHUNK, TW), _f32),
            pltpu.VMEM((10, CHUNK), _f32),
            pltpu.VMEM((NWAVE, CHUNK), _f32),
            pltpu.VMEM((ZROWS, SW), _f32),
            pltpu.VMEM_SHARED((NATOM, SW), _f32),
            pltpu.SemaphoreType.DMA,
        ],
    )
    return f(idx0, idx1, g1, gw, t)


def _density_groups(m):
    sq = m * m
    d0 = sq[:, 0:8]
    d1 = sq[:, 8:16] + sq[:, 16:24] + sq[:, 24:32]
    d2 = (sq[:, 32:40] + sq[:, 40:48] + sq[:, 48:56] + sq[:, 56:64]
          + sq[:, 64:72])
    return jnp.concatenate([d0, d1, d2], axis=1)


def _dens_mlp_body(sa, sb, dprev, w1, b1, w2, b2, w3, b3, dens_ref, t_ref):
    m = sa[:, :72] + sb[:, :72]
    dens = dprev[...] + _density_groups(m)
    dens_ref[...] = dens
    cnew = _mlp(dens, [(w1, b1), (w2, b2), (w3, b3)])
    t_ref[...] = jnp.concatenate([m, cnew], axis=1)


def _dens_copy_body(sa, sb, dprev, cin, dens_ref, t_ref):
    m = sa[:, :72] + sb[:, :72]
    dens_ref[...] = dprev[...] + _density_groups(m)
    t_ref[...] = jnp.concatenate([m, cin[...]], axis=1)


def _dens_step(sa, sb, dprev, mlp_params=None, cin=None):
    br = 2000
    extra_specs = []
    extra_vals = []
    if mlp_params is not None:
        body = _dens_mlp_body
        for w, b in mlp_params:
            extra_specs += [pl.BlockSpec(w.shape, lambda i: (0, 0)),
                            pl.BlockSpec((1, b.shape[0]), lambda i: (0, 0))]
            extra_vals += [w, b.reshape(1, -1)]
    else:
        body = _dens_copy_body
        extra_specs = [pl.BlockSpec((br, NWAVE), lambda i: (i, 0))]
        extra_vals = [cin]
    return pl.pallas_call(
        body,
        grid=(NATOM // br,),
        in_specs=[pl.BlockSpec((br, SW), lambda i: (i, 0)),
                  pl.BlockSpec((br, SW), lambda i: (i, 0)),
                  pl.BlockSpec((br, NORBIT), lambda i: (i, 0))] + extra_specs,
        out_specs=[pl.BlockSpec((br, NORBIT), lambda i: (i, 0)),
                   pl.BlockSpec((br, TW), lambda i: (i, 0))],
        out_shape=[jax.ShapeDtypeStruct((NATOM, NORBIT), _f32),
                   jax.ShapeDtypeStruct((NATOM, TW), _f32)],
    )(sa, sb, dprev, *extra_vals)


def _final_body(sa, sb, dprev, w1, b1, w2, b2, w3, b3, out_ref):
    m = sa[:, :72] + sb[:, :72]
    dens = dprev[...] + _density_groups(m)
    out_ref[...] = _mlp(dens, [(w1, b1), (w2, b2), (w3, b3)])


def _final(sa, sb, dprev, out_params):
    br = 2000
    wspecs = []
    wvals = []
    for w, b in out_params:
        wspecs += [pl.BlockSpec(w.shape, lambda i: (0, 0)),
                   pl.BlockSpec((1, b.shape[0]), lambda i: (0, 0))]
        wvals += [w, b.reshape(1, -1)]
    return pl.pallas_call(
        _final_body,
        grid=(NATOM // br,),
        in_specs=[pl.BlockSpec((br, SW), lambda i: (i, 0)),
                  pl.BlockSpec((br, SW), lambda i: (i, 0)),
                  pl.BlockSpec((br, NORBIT), lambda i: (i, 0))] + wspecs,
        out_specs=pl.BlockSpec((br, 1), lambda i: (i, 0)),
        out_shape=jax.ShapeDtypeStruct((NATOM, 1), _f32),
    )(sa, sb, dprev, *wvals)


def kernel(cart, atomindex, shifts, species, emb_params, mp_params, out_params):
    idx0 = atomindex[0]
    idx1 = atomindex[1]

    emb = _embed(species.reshape(-1, 1), emb_params)
    t0 = jnp.pad(emb, ((0, 0), (0, TW0 - NORBIT)))

    dxyz = _dxyz(cart, idx0, idx1)
    g0, g1 = _geom(dxyz, shifts)

    s0, gw = _pass0(idx0, idx1, g0, t0)
    dens0, t1 = _dens_step(s0[0], s0[1], jnp.zeros((NATOM, NORBIT), _f32),
                           mlp_params=mp_params[0])
    s1 = _passm(idx0, idx1, g1, gw, t1)
    dens1, t2 = _dens_step(s1[0], s1[1], dens0, cin=t1[:, 72:80])
    s2 = _passm(idx0, idx1, g1, gw, t2)
    out = _final(s2[0], s2[1], dens1, out_params)
    return out.reshape(-1)
